# Initial kernel scaffold; baseline (speedup 1.0000x reference)
import jax
import jax.numpy as jnp
from jax.experimental import pallas as pl


def kernel(x, W):
    raise NotImplementedError("write your pallas kernel here")


# speedup vs baseline: 1.1231x; 1.1231x over previous
import functools

import jax
import jax.numpy as jnp
from jax.experimental import pallas as pl
from jax.experimental.pallas import tpu as pltpu

TOKENS = 32768
HIDDEN = 4096
N_EXPERTS = 64
TOP_K = 8
TILE = 512


def _gate_kernel(x_ref, w_ref, probs_ref, vals_ref, idx_ref):
    x = x_ref[...]
    w = w_ref[...]
    logits = jax.lax.dot_general(
        x, w, (((1,), (1,)), ((), ())), preferred_element_type=jnp.float32
    )
    m = jnp.max(logits, axis=-1, keepdims=True)
    e = jnp.exp(logits - m)
    s = jnp.sum(e, axis=-1, keepdims=True)
    probs = e / s
    probs_ref[...] = probs

    work = probs
    iota = jax.lax.broadcasted_iota(jnp.int32, probs.shape, 1)
    vals = []
    idxs = []
    for _ in range(TOP_K):
        v = jnp.max(work, axis=-1, keepdims=True)
        i = jnp.min(
            jnp.where(work == v, iota, N_EXPERTS), axis=-1, keepdims=True
        )
        vals.append(v)
        idxs.append(i)
        work = jnp.where(iota == i, -jnp.inf, work)
    top_vals = jnp.concatenate(vals, axis=-1)
    top_idx = jnp.concatenate(idxs, axis=-1)
    top_vals = top_vals / jnp.sum(top_vals, axis=-1, keepdims=True)
    vals_ref[...] = top_vals
    idx_ref[...] = top_idx


@jax.jit
def kernel(x, W):
    n_tiles = TOKENS // TILE
    probs, top_vals, top_idx = pl.pallas_call(
        _gate_kernel,
        grid=(n_tiles,),
        in_specs=[
            pl.BlockSpec((TILE, HIDDEN), lambda i: (i, 0)),
            pl.BlockSpec((N_EXPERTS, HIDDEN), lambda i: (0, 0)),
        ],
        out_specs=[
            pl.BlockSpec((TILE, N_EXPERTS), lambda i: (i, 0)),
            pl.BlockSpec((TILE, TOP_K), lambda i: (i, 0)),
            pl.BlockSpec((TILE, TOP_K), lambda i: (i, 0)),
        ],
        out_shape=[
            jax.ShapeDtypeStruct((TOKENS, N_EXPERTS), jnp.float32),
            jax.ShapeDtypeStruct((TOKENS, TOP_K), jnp.float32),
            jax.ShapeDtypeStruct((TOKENS, TOP_K), jnp.int32),
        ],
        compiler_params=pltpu.CompilerParams(
            dimension_semantics=("arbitrary",),
        ),
    )(x, W)
    return (probs, top_vals, top_idx)


# speedup vs baseline: 1.1637x; 1.0362x over previous
import numpy as np

import jax
import jax.numpy as jnp
from jax.experimental import pallas as pl
from jax.experimental.pallas import tpu as pltpu

TOKENS = 32768
HIDDEN = 4096
N_EXPERTS = 64
TOP_K = 8
TILE = 512

_idx_np = np.zeros((N_EXPERTS * TOP_K, TOP_K), np.float32)
for _j in range(TOP_K):
    _idx_np[N_EXPERTS * _j : N_EXPERTS * (_j + 1), _j] = np.arange(N_EXPERTS)
_IDX_MAT = _idx_np


def _gate_kernel(x_ref, w_ref, s_ref, probs_ref, vals_ref, idx_ref):
    x = x_ref[...]
    w = w_ref[...]
    logits = jax.lax.dot_general(
        x, w, (((1,), (1,)), ((), ())), preferred_element_type=jnp.float32
    )
    m = jnp.max(logits, axis=-1, keepdims=True)
    e = jnp.exp(logits - m)
    s = jnp.sum(e, axis=-1, keepdims=True)
    probs = e / s
    probs_ref[...] = probs

    work = probs
    vals = []
    masks = []
    for _ in range(TOP_K):
        v = jnp.max(work, axis=-1, keepdims=True)
        hit = work == v
        vals.append(v)
        masks.append(jnp.where(hit, 1.0, 0.0))
        work = jnp.where(hit, -jnp.inf, work)
    top_vals = jnp.concatenate(vals, axis=-1)
    top_vals = top_vals / jnp.sum(top_vals, axis=-1, keepdims=True)
    vals_ref[...] = top_vals

    mask_cat = jnp.concatenate(masks, axis=-1)
    idx_f = jax.lax.dot_general(
        mask_cat,
        s_ref[...],
        (((1,), (0,)), ((), ())),
        preferred_element_type=jnp.float32,
    )
    idx_ref[...] = idx_f.astype(jnp.int32)


@jax.jit
def kernel(x, W):
    n_tiles = TOKENS // TILE
    idx_mat = jnp.asarray(_IDX_MAT)
    probs, top_vals, top_idx = pl.pallas_call(
        _gate_kernel,
        grid=(n_tiles,),
        in_specs=[
            pl.BlockSpec((TILE, HIDDEN), lambda i: (i, 0)),
            pl.BlockSpec((N_EXPERTS, HIDDEN), lambda i: (0, 0)),
            pl.BlockSpec((N_EXPERTS * TOP_K, TOP_K), lambda i: (0, 0)),
        ],
        out_specs=[
            pl.BlockSpec((TILE, N_EXPERTS), lambda i: (i, 0)),
            pl.BlockSpec((TILE, TOP_K), lambda i: (i, 0)),
            pl.BlockSpec((TILE, TOP_K), lambda i: (i, 0)),
        ],
        out_shape=[
            jax.ShapeDtypeStruct((TOKENS, N_EXPERTS), jnp.float32),
            jax.ShapeDtypeStruct((TOKENS, TOP_K), jnp.float32),
            jax.ShapeDtypeStruct((TOKENS, TOP_K), jnp.int32),
        ],
        compiler_params=pltpu.CompilerParams(
            dimension_semantics=("arbitrary",),
        ),
    )(x, W, idx_mat)
    return (probs, top_vals, top_idx)


# speedup vs baseline: 1.3992x; 1.2023x over previous
import jax
import jax.numpy as jnp
from jax.experimental import pallas as pl
from jax.experimental.pallas import tpu as pltpu

TOKENS = 32768
HIDDEN = 4096
N_EXPERTS = 64
TOP_K = 8
TILE = 1024


def _gate_kernel(x_ref, w_ref, probs_ref, vals_ref, idx_ref):
    x = x_ref[...]
    w = w_ref[...]
    logits = jax.lax.dot_general(
        x, w, (((1,), (1,)), ((), ())), preferred_element_type=jnp.float32
    )
    m = jnp.max(logits, axis=-1, keepdims=True)
    e = jnp.exp(logits - m)
    s = jnp.sum(e, axis=-1, keepdims=True)
    probs = e / s
    probs_ref[...] = probs

    work = probs
    iota = jax.lax.broadcasted_iota(jnp.int32, probs.shape, 1).astype(
        jnp.float32
    )
    vals = []
    idxs = []
    for _ in range(TOP_K):
        v = jnp.max(work, axis=-1, keepdims=True)
        i = jnp.min(
            jnp.where(work == v, iota, float(N_EXPERTS)),
            axis=-1,
            keepdims=True,
        )
        vals.append(v)
        idxs.append(i)
        work = jnp.where(iota == i, -jnp.inf, work)
    top_vals = jnp.concatenate(vals, axis=-1)
    top_idx = jnp.concatenate(idxs, axis=-1)
    top_vals = top_vals / jnp.sum(top_vals, axis=-1, keepdims=True)
    vals_ref[...] = top_vals
    idx_ref[...] = top_idx.astype(jnp.int32)


@jax.jit
def kernel(x, W):
    n_tiles = TOKENS // TILE
    probs, top_vals, top_idx = pl.pallas_call(
        _gate_kernel,
        grid=(n_tiles,),
        in_specs=[
            pl.BlockSpec((TILE, HIDDEN), lambda i: (i, 0)),
            pl.BlockSpec((N_EXPERTS, HIDDEN), lambda i: (0, 0)),
        ],
        out_specs=[
            pl.BlockSpec((TILE, N_EXPERTS), lambda i: (i, 0)),
            pl.BlockSpec((TILE, TOP_K), lambda i: (i, 0)),
            pl.BlockSpec((TILE, TOP_K), lambda i: (i, 0)),
        ],
        out_shape=[
            jax.ShapeDtypeStruct((TOKENS, N_EXPERTS), jnp.float32),
            jax.ShapeDtypeStruct((TOKENS, TOP_K), jnp.float32),
            jax.ShapeDtypeStruct((TOKENS, TOP_K), jnp.int32),
        ],
        compiler_params=pltpu.CompilerParams(
            dimension_semantics=("parallel",),
        ),
    )(x, W)
    return (probs, top_vals, top_idx)
